# baseline (speedup 1.0000x reference)
import functools

import jax
import jax.numpy as jnp
from jax import lax
from jax.experimental import pallas as pl
from jax.experimental.pallas import tpu as pltpu
from jax.experimental.pallas import tpu_sc as plsc

NC, NS = 2, 16
NW = NC * NS
CH = 128

BE = 4000
BN = 2000


_SC_PARAMS = pltpu.CompilerParams(use_tc_tiling_on_sc=False)


def _relu(v):
    return jnp.maximum(v, 0.0)


def _mm(a, b):
    return lax.dot_general(a, b, (((1,), (0,)), ((), ())),
                           preferred_element_type=jnp.float32)


def _bcast_spec(shape):
    return pl.BlockSpec(shape, lambda i: tuple(0 for _ in shape))


def _edge_enc_body(ea, w1, b1, w2, b2, w3, b3, out):
    h = _relu(_mm(ea[...], w1[...]) + b1[...])
    h = _relu(_mm(h, w2[...]) + b2[...])
    out[...] = _relu(_mm(h, w3[...]) + b3[...])


def _edge_encode(edge_attr, w1, b1, w2, b2, w3, b3):
    E, F = edge_attr.shape
    grid = E // BE
    return pl.pallas_call(
        _edge_enc_body,
        grid=(grid,),
        in_specs=[
            pl.BlockSpec((BE, F), lambda i: (i, 0)),
            _bcast_spec(w1.shape), _bcast_spec((1, b1.shape[0])),
            _bcast_spec(w2.shape), _bcast_spec((1, b2.shape[0])),
            _bcast_spec(w3.shape), _bcast_spec((1, b3.shape[0])),
        ],
        out_specs=pl.BlockSpec((BE, w3.shape[1]), lambda i: (i, 0)),
        out_shape=jax.ShapeDtypeStruct((E, w3.shape[1]), jnp.float32),
    )(edge_attr, w1, b1.reshape(1, -1), w2, b2.reshape(1, -1),
      w3, b3.reshape(1, -1))


def _node_prep_body(x, xl, n1w, n1b, n2w, n2b, bm, cm, bn, w1a, w1b,
                    h_out, base_out, g1_out, g2_out):
    h = _relu(_mm(x[...], n1w[...]) + n1b[...])
    h = _relu(_mm(h, n2w[...]) + n2b[...])
    h_out[...] = h
    base_out[...] = _mm(xl[...], bm[...]) + _mm(h, cm[...]) + bn[...]
    g1_out[...] = _mm(h, w1a[...])
    g2_out[...] = _mm(h, w1b[...])


def _node_prep(x, x_lstm, n1w, n1b, n2w, n2b, bm, cm, bn, w1a, w1b):
    N, F = x.shape
    D = x_lstm.shape[1]
    H = w1a.shape[1]
    grid = N // BN
    return pl.pallas_call(
        _node_prep_body,
        grid=(grid,),
        in_specs=[
            pl.BlockSpec((BN, F), lambda i: (i, 0)),
            pl.BlockSpec((BN, D), lambda i: (i, 0)),
            _bcast_spec(n1w.shape), _bcast_spec((1, n1b.shape[0])),
            _bcast_spec(n2w.shape), _bcast_spec((1, n2b.shape[0])),
            _bcast_spec(bm.shape), _bcast_spec(cm.shape),
            _bcast_spec((1, bn.shape[0])),
            _bcast_spec(w1a.shape), _bcast_spec(w1b.shape),
        ],
        out_specs=[
            pl.BlockSpec((BN, D), lambda i: (i, 0)),
            pl.BlockSpec((BN, D), lambda i: (i, 0)),
            pl.BlockSpec((BN, H), lambda i: (i, 0)),
            pl.BlockSpec((BN, H), lambda i: (i, 0)),
        ],
        out_shape=[
            jax.ShapeDtypeStruct((N, D), jnp.float32),
            jax.ShapeDtypeStruct((N, D), jnp.float32),
            jax.ShapeDtypeStruct((N, H), jnp.float32),
            jax.ShapeDtypeStruct((N, H), jnp.float32),
        ],
    )(x, x_lstm, n1w, n1b.reshape(1, -1), n2w, n2b.reshape(1, -1),
      bm, cm, bn.reshape(1, -1), w1a, w1b)


def _edge_mlp_body(gs, e, w1c, b1, w2, b2, out):
    u = _relu(gs[...] + _mm(e[...], w1c[...]) + b1[...])
    out[...] = _relu(_mm(u, w2[...]) + b2[...])


def _edge_mlp(gs, e, w1c, b1, w2, b2):
    E, H = gs.shape
    D = e.shape[1]
    grid = E // BE
    return pl.pallas_call(
        _edge_mlp_body,
        grid=(grid,),
        in_specs=[
            pl.BlockSpec((BE, H), lambda i: (i, 0)),
            pl.BlockSpec((BE, D), lambda i: (i, 0)),
            _bcast_spec(w1c.shape), _bcast_spec((1, b1.shape[0])),
            _bcast_spec(w2.shape), _bcast_spec((1, b2.shape[0])),
        ],
        out_specs=pl.BlockSpec((BE, D), lambda i: (i, 0)),
        out_shape=jax.ShapeDtypeStruct((E, D), jnp.float32),
    )(gs, e, w1c, b1.reshape(1, -1), w2, b2.reshape(1, -1))


def _edge_mlp0_body(gs, ea, ew1, eb1, ew2, eb2, ew3, eb3,
                    w1c, b1, w2, b2, out):
    e0 = _relu(_mm(ea[...], ew1[...]) + eb1[...])
    e0 = _relu(_mm(e0, ew2[...]) + eb2[...])
    e0 = _relu(_mm(e0, ew3[...]) + eb3[...])
    u = _relu(gs[...] + _mm(e0, w1c[...]) + b1[...])
    out[...] = _relu(_mm(u, w2[...]) + b2[...])


def _edge_mlp0(gs, edge_attr, ew1, eb1, ew2, eb2, ew3, eb3, w1c, b1, w2, b2):
    E, H = gs.shape
    F = edge_attr.shape[1]
    grid = E // BE
    return pl.pallas_call(
        _edge_mlp0_body,
        grid=(grid,),
        in_specs=[
            pl.BlockSpec((BE, H), lambda i: (i, 0)),
            pl.BlockSpec((BE, F), lambda i: (i, 0)),
            _bcast_spec(ew1.shape), _bcast_spec((1, eb1.shape[0])),
            _bcast_spec(ew2.shape), _bcast_spec((1, eb2.shape[0])),
            _bcast_spec(ew3.shape), _bcast_spec((1, eb3.shape[0])),
            _bcast_spec(w1c.shape), _bcast_spec((1, b1.shape[0])),
            _bcast_spec(w2.shape), _bcast_spec((1, b2.shape[0])),
        ],
        out_specs=pl.BlockSpec((BE, w2.shape[1]), lambda i: (i, 0)),
        out_shape=jax.ShapeDtypeStruct((E, w2.shape[1]), jnp.float32),
    )(gs, edge_attr, ew1, eb1.reshape(1, -1), ew2, eb2.reshape(1, -1),
      ew3, eb3.reshape(1, -1), w1c, b1.reshape(1, -1), w2, b2.reshape(1, -1))


def _node_update_body(h, p0, p1, base, am, dm, w1a, w1b,
                      h_out, g1_out, g2_out):
    agg = p0[...] + p1[...]
    hn = _relu(_mm(h[...], am[...]) + _mm(agg, dm[...]) + base[...])
    h_out[...] = hn
    g1_out[...] = _mm(hn, w1a[...])
    g2_out[...] = _mm(hn, w1b[...])


def _node_update(h, p0, p1, base, am, dm, w1a, w1b):
    N, D = h.shape
    P = p0.shape[1]
    H = w1a.shape[1]
    grid = N // BN
    blk = lambda w: pl.BlockSpec((BN, w), lambda i: (i, 0))
    return pl.pallas_call(
        _node_update_body,
        grid=(grid,),
        in_specs=[blk(D), blk(P), blk(P), blk(D),
                  _bcast_spec(am.shape), _bcast_spec(dm.shape),
                  _bcast_spec(w1a.shape), _bcast_spec(w1b.shape)],
        out_specs=[blk(D), blk(H), blk(H)],
        out_shape=[
            jax.ShapeDtypeStruct((N, D), jnp.float32),
            jax.ShapeDtypeStruct((N, H), jnp.float32),
            jax.ShapeDtypeStruct((N, H), jnp.float32),
        ],
    )(h, p0, p1, base, am, dm, w1a, w1b)


def _node_final_body(h, p0, p1, base, am, dm,
                     r1w, r1b, r2w, r2b, r3w, r3b, r4w, r4b, out):
    agg = p0[...] + p1[...]
    hn = _relu(_mm(h[...], am[...]) + _mm(agg, dm[...]) + base[...])
    o = _relu(_mm(hn, r1w[...]) + r1b[...])
    o = _relu(_mm(o, r2w[...]) + r2b[...])
    o = _relu(_mm(o, r3w[...]) + r3b[...])
    out[...] = _mm(o, r4w[...]) + r4b[...]


def _node_final(h, p0, p1, base, am, dm, r1w, r1b, r2w, r2b, r3w, r3b,
                r4w, r4b):
    N, D = h.shape
    P = p0.shape[1]
    O = r4w.shape[1]
    grid = N // BN
    blk = lambda w: pl.BlockSpec((BN, w), lambda i: (i, 0))
    return pl.pallas_call(
        _node_final_body,
        grid=(grid,),
        in_specs=[blk(D), blk(P), blk(P), blk(D),
                  _bcast_spec(am.shape), _bcast_spec(dm.shape),
                  _bcast_spec(r1w.shape), _bcast_spec((1, r1b.shape[0])),
                  _bcast_spec(r2w.shape), _bcast_spec((1, r2b.shape[0])),
                  _bcast_spec(r3w.shape), _bcast_spec((1, r3b.shape[0])),
                  _bcast_spec(r4w.shape), _bcast_spec((1, r4b.shape[0]))],
        out_specs=blk(O),
        out_shape=jax.ShapeDtypeStruct((N, O), jnp.float32),
    )(h, p0, p1, base, am, dm, r1w, r1b.reshape(1, -1), r2w,
      r2b.reshape(1, -1), r3w, r3b.reshape(1, -1), r4w, r4b.reshape(1, -1))


def _worker_range(wid, nch):
    per = (nch + NW - 1) // NW
    extra = nch - (per - 1) * NW
    start = wid * (per - 1) + jnp.minimum(wid, extra)
    cnt = (per - 1) + (wid < extra).astype(jnp.int32)
    lbase = jnp.minimum(start, nch - per)
    loff = start - lbase
    return per, start, cnt, lbase, loff


def _sc_gather(g1, g2, row2d, col2d, width):
    Np, K = g1.shape
    nch = row2d.shape[0]
    mesh = plsc.VectorSubcoreMesh(core_axis_name="c", subcore_axis_name="s",
                                  num_cores=NC, num_subcores=NS)
    ncol = width // 16

    def body(g1_hbm, g2_hbm, row_hbm, col_hbm, o_hbm,
             idxr, idxc, buf1, buf2, gsem, wsem):
        wid = lax.axis_index("s") * NC + lax.axis_index("c")
        per, start, cnt, lbase, loff = _worker_range(wid, nch)
        pltpu.sync_copy(row_hbm.at[pl.ds(lbase, per), :], idxr)
        pltpu.sync_copy(col_hbm.at[pl.ds(lbase, per), :], idxc)

        def fire(j, b):
            pltpu.async_copy(g1_hbm.at[idxr.at[loff + j]], buf1.at[b], gsem)
            pltpu.async_copy(g2_hbm.at[idxc.at[loff + j]], buf2.at[b], gsem)

        def wait_gathers(b):
            pltpu.make_async_copy(g1_hbm.at[idxr.at[loff]], buf1.at[b],
                                  gsem).wait()
            pltpu.make_async_copy(g2_hbm.at[idxc.at[loff]], buf2.at[b],
                                  gsem).wait()

        def add_rows(b):
            def row_add(r, carry):
                for c in range(ncol):
                    sl = pl.ds(c * 16, 16)
                    buf1[b, r, sl] = buf1[b, r, sl] + buf2[b, r, sl]
                return carry
            lax.fori_loop(0, CH, row_add, 0)

        def writeback(j, b):
            base = pl.multiple_of((start + j) * CH, CH)
            pltpu.async_copy(buf1.at[b], o_hbm.at[pl.ds(base, CH), :], wsem)

        def wait_writeback(b):
            base = pl.multiple_of(start * CH, CH)
            pltpu.make_async_copy(buf1.at[b], o_hbm.at[pl.ds(base, CH), :],
                                  wsem).wait()

        fire(0, 0)

        def step(j, carry):
            b = lax.rem(j, 2)
            nb = 1 - b
            wait_gathers(b)

            @pl.when(j + 1 < cnt)
            def _():
                @pl.when(j >= 1)
                def _():
                    wait_writeback(nb)
                fire(j + 1, nb)

            add_rows(b)
            writeback(j, b)
            return carry

        lax.fori_loop(0, cnt, step, 0)
        wait_writeback(0)
        wait_writeback(1)

    f = pl.kernel(
        body,
        out_type=jax.ShapeDtypeStruct((nch * CH, K), jnp.float32),
        mesh=mesh,
        scratch_types=[
            pltpu.VMEM(((nch + NW - 1) // NW, CH), jnp.int32),
            pltpu.VMEM(((nch + NW - 1) // NW, CH), jnp.int32),
            pltpu.VMEM((2, CH, K), jnp.float32),
            pltpu.VMEM((2, CH, K), jnp.float32),
            pltpu.SemaphoreType.DMA,
            pltpu.SemaphoreType.DMA,
        ],
        compiler_params=_SC_PARAMS,
    )
    return f(g1, g2, row2d, col2d)


def _sc_scatter(e, row2d, zeros):
    E_, D = e.shape
    Np = zeros.shape[0]
    nch = row2d.shape[0]
    per = (nch + NW - 1) // NW
    npt = Np // NS
    mesh = plsc.VectorSubcoreMesh(core_axis_name="c", subcore_axis_name="s", num_cores=NC, num_subcores=NS)

    def body(e_hbm, row_hbm, z_hbm, out_hbm, idxb, ebuf, acc, lsem, ssem):
        cidx = lax.axis_index("c")
        sidx = lax.axis_index("s")
        wid = sidx * NC + cidx
        per_, start, cnt, lbase, loff = _worker_range(wid, nch)
        rbase = pl.multiple_of(sidx * npt, npt)
        zcp = pltpu.async_copy(z_hbm.at[pl.ds(rbase, npt), :],
                               acc.at[pl.ds(rbase, npt), :], lsem)
        pltpu.sync_copy(row_hbm.at[pl.ds(lbase, per_), :], idxb)
        zcp.wait()
        plsc.subcore_barrier()

        def load(j, b):
            base = pl.multiple_of((start + j) * CH, CH)
            pltpu.async_copy(e_hbm.at[pl.ds(base, CH), :], ebuf.at[b], lsem)

        def wait_load(b):
            base = pl.multiple_of(start * CH, CH)
            pltpu.make_async_copy(e_hbm.at[pl.ds(base, CH), :], ebuf.at[b],
                                  lsem).wait()

        def scat(j, b):
            pltpu.async_copy(ebuf.at[b], acc.at[idxb.at[loff + j]], ssem,
                             add=True)

        def wait_scat(b):
            pltpu.make_async_copy(ebuf.at[b], acc.at[idxb.at[loff]],
                                  ssem).wait()

        load(0, 0)

        def step(j, carry):
            b = lax.rem(j, 2)
            nb = 1 - b
            wait_load(b)

            @pl.when(j + 1 < cnt)
            def _():
                @pl.when(j >= 1)
                def _():
                    wait_scat(nb)
                load(j + 1, nb)

            scat(j, b)
            return carry

        lax.fori_loop(0, cnt, step, 0)
        wait_scat(0)
        wait_scat(1)
        plsc.subcore_barrier()
        pltpu.sync_copy(acc.at[pl.ds(rbase, npt), :],
                        out_hbm.at[cidx, pl.ds(rbase, npt), :])

    f = pl.kernel(
        body,
        out_type=jax.ShapeDtypeStruct((NC, Np, D), jnp.float32),
        mesh=mesh,
        scratch_types=[
            pltpu.VMEM((per, CH), jnp.int32),
            pltpu.VMEM((2, CH, D), jnp.float32),
            pltpu.VMEM_SHARED((Np, D), jnp.float32),
            pltpu.SemaphoreType.DMA,
            pltpu.SemaphoreType.DMA,
        ],
        compiler_params=_SC_PARAMS,
    )
    return f(e, row2d, zeros)


def kernel(x, x_lstm, edge_attr, edge_index,
           ee1_w, ee1_b, ee2_w, ee2_b, ee3_w, ee3_b,
           ne1_w, ne1_b, ne2_w, ne2_b,
           me1_w, me1_b, me2_w, me2_b,
           mn1_w, mn1_b,
           r1_w, r1_b, r2_w, r2_b, r3_w, r3_b, r4_w, r4_b):
    N = x.shape[0]
    E = edge_attr.shape[0]
    D = x_lstm.shape[1]
    T = 3
    assert E % CH == 0 and E % BE == 0 and N % BN == 0 and N % NS == 0

    row2d = edge_index[0].reshape(E // CH, CH)
    col2d = edge_index[1].reshape(E // CH, CH)

    H = me1_w.shape[1]
    w1a, w1b, w1c = me1_w[:D], me1_w[D:2 * D], me1_w[2 * D:]
    am, bm, cm, dm = (mn1_w[:D], mn1_w[D:2 * D],
                      mn1_w[2 * D:3 * D], mn1_w[3 * D:])

    P = 128
    pad_c = lambda w: jnp.pad(w, ((0, 0), (0, P - w.shape[1])))
    pad_v = lambda b: jnp.pad(b, (0, P - b.shape[0]))
    w1a_p, w1b_p = pad_c(w1a), pad_c(w1b)
    w1c_p = jnp.pad(w1c, ((0, P - w1c.shape[0]), (0, P - w1c.shape[1])))
    me2_p = jnp.pad(me2_w, ((0, P - me2_w.shape[0]), (0, P - me2_w.shape[1])))
    dm_p = jnp.pad(dm, ((0, P - dm.shape[0]), (0, 0)))

    h, base, g1, g2 = _node_prep(x, x_lstm, ne1_w, ne1_b, ne2_w, ne2_b,
                                 bm, cm, mn1_b, w1a_p, w1b_p)
    zeros = jnp.zeros((N, P), jnp.float32)
    b1_p, b2_p = pad_v(me1_b), pad_v(me2_b)

    e = None
    out = None
    for t in range(T):
        gs = _sc_gather(g1, g2, row2d, col2d, H)
        if t == 0:
            e = _edge_mlp0(gs, edge_attr, ee1_w, ee1_b, ee2_w, ee2_b,
                           pad_c(ee3_w), pad_v(ee3_b), w1c_p, b1_p,
                           me2_p, b2_p)
        else:
            e = _edge_mlp(gs, e, w1c_p, b1_p, me2_p, b2_p)
        parts = _sc_scatter(e, row2d, zeros)
        if t < T - 1:
            h, g1, g2 = _node_update(h, parts[0], parts[1], base, am, dm_p,
                                     w1a_p, w1b_p)
        else:
            out = _node_final(h, parts[0], parts[1], base, am, dm_p,
                              r1_w, r1_b, r2_w, r2_b, r3_w, r3_b, r4_w, r4_b)
    return out


# speedup vs baseline: 1.0704x; 1.0704x over previous
import functools

import jax
import jax.numpy as jnp
from jax import lax
from jax.experimental import pallas as pl
from jax.experimental.pallas import tpu as pltpu
from jax.experimental.pallas import tpu_sc as plsc

NC, NS = 2, 16
NW = NC * NS
CH = 128

BE = 4000
BN = 2000


_SC_PARAMS = pltpu.CompilerParams(use_tc_tiling_on_sc=False)


def _relu(v):
    return jnp.maximum(v, 0.0)


def _mm(a, b):
    return lax.dot_general(a, b, (((1,), (0,)), ((), ())),
                           preferred_element_type=jnp.float32)


def _bcast_spec(shape):
    return pl.BlockSpec(shape, lambda i: tuple(0 for _ in shape))


def _edge_enc_body(ea, w1, b1, w2, b2, w3, b3, out):
    h = _relu(_mm(ea[...], w1[...]) + b1[...])
    h = _relu(_mm(h, w2[...]) + b2[...])
    out[...] = _relu(_mm(h, w3[...]) + b3[...])


def _edge_encode(edge_attr, w1, b1, w2, b2, w3, b3):
    E, F = edge_attr.shape
    grid = E // BE
    return pl.pallas_call(
        _edge_enc_body,
        grid=(grid,),
        in_specs=[
            pl.BlockSpec((BE, F), lambda i: (i, 0)),
            _bcast_spec(w1.shape), _bcast_spec((1, b1.shape[0])),
            _bcast_spec(w2.shape), _bcast_spec((1, b2.shape[0])),
            _bcast_spec(w3.shape), _bcast_spec((1, b3.shape[0])),
        ],
        out_specs=pl.BlockSpec((BE, w3.shape[1]), lambda i: (i, 0)),
        out_shape=jax.ShapeDtypeStruct((E, w3.shape[1]), jnp.float32),
    )(edge_attr, w1, b1.reshape(1, -1), w2, b2.reshape(1, -1),
      w3, b3.reshape(1, -1))


def _node_prep_body(x, xl, n1w, n1b, n2w, n2b, bm, cm, bn, w1a, w1b,
                    h_out, base_out, g1_out, g2_out):
    h = _relu(_mm(x[...], n1w[...]) + n1b[...])
    h = _relu(_mm(h, n2w[...]) + n2b[...])
    h_out[...] = h
    base_out[...] = _mm(xl[...], bm[...]) + _mm(h, cm[...]) + bn[...]
    g1_out[...] = _mm(h, w1a[...])
    g2_out[...] = _mm(h, w1b[...])


def _node_prep(x, x_lstm, n1w, n1b, n2w, n2b, bm, cm, bn, w1a, w1b):
    N, F = x.shape
    D = x_lstm.shape[1]
    H = w1a.shape[1]
    grid = N // BN
    return pl.pallas_call(
        _node_prep_body,
        grid=(grid,),
        in_specs=[
            pl.BlockSpec((BN, F), lambda i: (i, 0)),
            pl.BlockSpec((BN, D), lambda i: (i, 0)),
            _bcast_spec(n1w.shape), _bcast_spec((1, n1b.shape[0])),
            _bcast_spec(n2w.shape), _bcast_spec((1, n2b.shape[0])),
            _bcast_spec(bm.shape), _bcast_spec(cm.shape),
            _bcast_spec((1, bn.shape[0])),
            _bcast_spec(w1a.shape), _bcast_spec(w1b.shape),
        ],
        out_specs=[
            pl.BlockSpec((BN, D), lambda i: (i, 0)),
            pl.BlockSpec((BN, D), lambda i: (i, 0)),
            pl.BlockSpec((BN, H), lambda i: (i, 0)),
            pl.BlockSpec((BN, H), lambda i: (i, 0)),
        ],
        out_shape=[
            jax.ShapeDtypeStruct((N, D), jnp.float32),
            jax.ShapeDtypeStruct((N, D), jnp.float32),
            jax.ShapeDtypeStruct((N, H), jnp.float32),
            jax.ShapeDtypeStruct((N, H), jnp.float32),
        ],
    )(x, x_lstm, n1w, n1b.reshape(1, -1), n2w, n2b.reshape(1, -1),
      bm, cm, bn.reshape(1, -1), w1a, w1b)


def _edge_mlp_body(gs, e, w1c, b1, w2, b2, out):
    u = _relu(gs[...] + _mm(e[...], w1c[...]) + b1[...])
    out[...] = _relu(_mm(u, w2[...]) + b2[...])


def _edge_mlp(gs, e, w1c, b1, w2, b2):
    E, H = gs.shape
    D = e.shape[1]
    grid = E // BE
    return pl.pallas_call(
        _edge_mlp_body,
        grid=(grid,),
        in_specs=[
            pl.BlockSpec((BE, H), lambda i: (i, 0)),
            pl.BlockSpec((BE, D), lambda i: (i, 0)),
            _bcast_spec(w1c.shape), _bcast_spec((1, b1.shape[0])),
            _bcast_spec(w2.shape), _bcast_spec((1, b2.shape[0])),
        ],
        out_specs=pl.BlockSpec((BE, D), lambda i: (i, 0)),
        out_shape=jax.ShapeDtypeStruct((E, D), jnp.float32),
    )(gs, e, w1c, b1.reshape(1, -1), w2, b2.reshape(1, -1))


def _edge_mlp0_body(gs, ea, ew1, eb1, ew2, eb2, ew3, eb3,
                    w1c, b1, w2, b2, out):
    e0 = _relu(_mm(ea[...], ew1[...]) + eb1[...])
    e0 = _relu(_mm(e0, ew2[...]) + eb2[...])
    e0 = _relu(_mm(e0, ew3[...]) + eb3[...])
    u = _relu(gs[...] + _mm(e0, w1c[...]) + b1[...])
    out[...] = _relu(_mm(u, w2[...]) + b2[...])


def _edge_mlp0(gs, edge_attr, ew1, eb1, ew2, eb2, ew3, eb3, w1c, b1, w2, b2):
    E, H = gs.shape
    F = edge_attr.shape[1]
    grid = E // BE
    return pl.pallas_call(
        _edge_mlp0_body,
        grid=(grid,),
        in_specs=[
            pl.BlockSpec((BE, H), lambda i: (i, 0)),
            pl.BlockSpec((BE, F), lambda i: (i, 0)),
            _bcast_spec(ew1.shape), _bcast_spec((1, eb1.shape[0])),
            _bcast_spec(ew2.shape), _bcast_spec((1, eb2.shape[0])),
            _bcast_spec(ew3.shape), _bcast_spec((1, eb3.shape[0])),
            _bcast_spec(w1c.shape), _bcast_spec((1, b1.shape[0])),
            _bcast_spec(w2.shape), _bcast_spec((1, b2.shape[0])),
        ],
        out_specs=pl.BlockSpec((BE, w2.shape[1]), lambda i: (i, 0)),
        out_shape=jax.ShapeDtypeStruct((E, w2.shape[1]), jnp.float32),
    )(gs, edge_attr, ew1, eb1.reshape(1, -1), ew2, eb2.reshape(1, -1),
      ew3, eb3.reshape(1, -1), w1c, b1.reshape(1, -1), w2, b2.reshape(1, -1))


def _node_update_body(h, p0, p1, base, am, dm, w1a, w1b,
                      h_out, g1_out, g2_out):
    agg = p0[...] + p1[...]
    hn = _relu(_mm(h[...], am[...]) + _mm(agg, dm[...]) + base[...])
    h_out[...] = hn
    g1_out[...] = _mm(hn, w1a[...])
    g2_out[...] = _mm(hn, w1b[...])


def _node_update(h, p0, p1, base, am, dm, w1a, w1b):
    N, D = h.shape
    P = p0.shape[1]
    H = w1a.shape[1]
    grid = N // BN
    blk = lambda w: pl.BlockSpec((BN, w), lambda i: (i, 0))
    return pl.pallas_call(
        _node_update_body,
        grid=(grid,),
        in_specs=[blk(D), blk(P), blk(P), blk(D),
                  _bcast_spec(am.shape), _bcast_spec(dm.shape),
                  _bcast_spec(w1a.shape), _bcast_spec(w1b.shape)],
        out_specs=[blk(D), blk(H), blk(H)],
        out_shape=[
            jax.ShapeDtypeStruct((N, D), jnp.float32),
            jax.ShapeDtypeStruct((N, H), jnp.float32),
            jax.ShapeDtypeStruct((N, H), jnp.float32),
        ],
    )(h, p0, p1, base, am, dm, w1a, w1b)


def _node_final_body(h, p0, p1, base, am, dm,
                     r1w, r1b, r2w, r2b, r3w, r3b, r4w, r4b, out):
    agg = p0[...] + p1[...]
    hn = _relu(_mm(h[...], am[...]) + _mm(agg, dm[...]) + base[...])
    o = _relu(_mm(hn, r1w[...]) + r1b[...])
    o = _relu(_mm(o, r2w[...]) + r2b[...])
    o = _relu(_mm(o, r3w[...]) + r3b[...])
    out[...] = _mm(o, r4w[...]) + r4b[...]


def _node_final(h, p0, p1, base, am, dm, r1w, r1b, r2w, r2b, r3w, r3b,
                r4w, r4b):
    N, D = h.shape
    P = p0.shape[1]
    O = r4w.shape[1]
    grid = N // BN
    blk = lambda w: pl.BlockSpec((BN, w), lambda i: (i, 0))
    return pl.pallas_call(
        _node_final_body,
        grid=(grid,),
        in_specs=[blk(D), blk(P), blk(P), blk(D),
                  _bcast_spec(am.shape), _bcast_spec(dm.shape),
                  _bcast_spec(r1w.shape), _bcast_spec((1, r1b.shape[0])),
                  _bcast_spec(r2w.shape), _bcast_spec((1, r2b.shape[0])),
                  _bcast_spec(r3w.shape), _bcast_spec((1, r3b.shape[0])),
                  _bcast_spec(r4w.shape), _bcast_spec((1, r4b.shape[0]))],
        out_specs=blk(O),
        out_shape=jax.ShapeDtypeStruct((N, O), jnp.float32),
    )(h, p0, p1, base, am, dm, r1w, r1b.reshape(1, -1), r2w,
      r2b.reshape(1, -1), r3w, r3b.reshape(1, -1), r4w, r4b.reshape(1, -1))


def _worker_range(wid, nch):
    per = (nch + NW - 1) // NW
    extra = nch - (per - 1) * NW
    start = wid * (per - 1) + jnp.minimum(wid, extra)
    cnt = (per - 1) + (wid < extra).astype(jnp.int32)
    lbase = jnp.minimum(start, nch - per)
    loff = start - lbase
    return per, start, cnt, lbase, loff


def _sc_gather(g1, g2, row2d, col2d, width):
    Np, K = g1.shape
    nch = row2d.shape[0]
    mesh = plsc.VectorSubcoreMesh(core_axis_name="c", subcore_axis_name="s",
                                  num_cores=NC, num_subcores=NS)
    ncol = width // 16

    def body(g1_hbm, g2_hbm, row_hbm, col_hbm, o_hbm,
             idxr, idxc, buf1, buf2, gsem, wsem):
        wid = lax.axis_index("s") * NC + lax.axis_index("c")
        per, start, cnt, lbase, loff = _worker_range(wid, nch)
        pltpu.sync_copy(row_hbm.at[pl.ds(lbase, per), :], idxr)
        pltpu.sync_copy(col_hbm.at[pl.ds(lbase, per), :], idxc)

        def fire(j, b):
            pltpu.async_copy(g1_hbm.at[idxr.at[loff + j]], buf1.at[b], gsem)
            pltpu.async_copy(g2_hbm.at[idxc.at[loff + j]], buf2.at[b], gsem)

        def wait_gathers(b):
            pltpu.make_async_copy(g1_hbm.at[idxr.at[loff]], buf1.at[b],
                                  gsem).wait()
            pltpu.make_async_copy(g2_hbm.at[idxc.at[loff]], buf2.at[b],
                                  gsem).wait()

        def add_rows(b):
            def row_add(i, carry):
                for u in range(8):
                    r = i * 8 + u
                    for c in range(ncol):
                        sl = pl.ds(c * 16, 16)
                        buf1[b, r, sl] = buf1[b, r, sl] + buf2[b, r, sl]
                return carry
            lax.fori_loop(0, CH // 8, row_add, 0)

        def writeback(j, b):
            base = pl.multiple_of((start + j) * CH, CH)
            pltpu.async_copy(buf1.at[b], o_hbm.at[pl.ds(base, CH), :], wsem)

        def wait_writeback(b):
            base = pl.multiple_of(start * CH, CH)
            pltpu.make_async_copy(buf1.at[b], o_hbm.at[pl.ds(base, CH), :],
                                  wsem).wait()

        fire(0, 0)

        def step(j, carry):
            b = lax.rem(j, 2)
            nb = 1 - b
            wait_gathers(b)

            @pl.when(j + 1 < cnt)
            def _():
                @pl.when(j >= 1)
                def _():
                    wait_writeback(nb)
                fire(j + 1, nb)

            add_rows(b)
            writeback(j, b)
            return carry

        lax.fori_loop(0, cnt, step, 0)
        wait_writeback(0)
        wait_writeback(1)

    f = pl.kernel(
        body,
        out_type=jax.ShapeDtypeStruct((nch * CH, K), jnp.float32),
        mesh=mesh,
        scratch_types=[
            pltpu.VMEM(((nch + NW - 1) // NW, CH), jnp.int32),
            pltpu.VMEM(((nch + NW - 1) // NW, CH), jnp.int32),
            pltpu.VMEM((2, CH, K), jnp.float32),
            pltpu.VMEM((2, CH, K), jnp.float32),
            pltpu.SemaphoreType.DMA,
            pltpu.SemaphoreType.DMA,
        ],
        compiler_params=_SC_PARAMS,
    )
    return f(g1, g2, row2d, col2d)


def _sc_scatter(e, row2d, zeros, width):
    E_, D = e.shape
    Np = zeros.shape[0]
    nch = row2d.shape[0]
    per = (nch + NW - 1) // NW
    npt = Np // NS
    mesh = plsc.VectorSubcoreMesh(core_axis_name="c", subcore_axis_name="s", num_cores=NC, num_subcores=NS)

    def body(e_hbm, row_hbm, z_hbm, out_hbm, idxb, ebuf, acc, lsem, ssem):
        cidx = lax.axis_index("c")
        sidx = lax.axis_index("s")
        wid = sidx * NC + cidx
        per_, start, cnt, lbase, loff = _worker_range(wid, nch)
        rbase = pl.multiple_of(sidx * npt, npt)
        zcp = pltpu.async_copy(z_hbm.at[pl.ds(rbase, npt), :],
                               acc.at[pl.ds(rbase, npt), :], lsem)
        pltpu.sync_copy(row_hbm.at[pl.ds(lbase, per_), :], idxb)
        zcp.wait()
        plsc.subcore_barrier()

        def load(j, b):
            base = pl.multiple_of((start + j) * CH, CH)
            pltpu.async_copy(e_hbm.at[pl.ds(base, CH), pl.ds(0, width)],
                             ebuf.at[b], lsem)

        def wait_load(b):
            base = pl.multiple_of(start * CH, CH)
            pltpu.make_async_copy(e_hbm.at[pl.ds(base, CH), pl.ds(0, width)],
                                  ebuf.at[b], lsem).wait()

        def scat(j, b):
            pltpu.async_copy(ebuf.at[b], acc.at[idxb.at[loff + j]], ssem,
                             add=True)

        def wait_scat(b):
            pltpu.make_async_copy(ebuf.at[b], acc.at[idxb.at[loff]],
                                  ssem).wait()

        load(0, 0)

        def step(j, carry):
            b = lax.rem(j, 2)
            nb = 1 - b
            wait_load(b)

            @pl.when(j + 1 < cnt)
            def _():
                @pl.when(j >= 1)
                def _():
                    wait_scat(nb)
                load(j + 1, nb)

            scat(j, b)
            return carry

        lax.fori_loop(0, cnt, step, 0)
        wait_scat(0)
        wait_scat(1)
        plsc.subcore_barrier()
        pltpu.sync_copy(acc.at[pl.ds(rbase, npt), :],
                        out_hbm.at[cidx, pl.ds(rbase, npt), :])

    f = pl.kernel(
        body,
        out_type=jax.ShapeDtypeStruct((NC, Np, width), jnp.float32),
        mesh=mesh,
        scratch_types=[
            pltpu.VMEM((per, CH), jnp.int32),
            pltpu.VMEM((2, CH, width), jnp.float32),
            pltpu.VMEM_SHARED((Np, width), jnp.float32),
            pltpu.SemaphoreType.DMA,
            pltpu.SemaphoreType.DMA,
        ],
        compiler_params=_SC_PARAMS,
    )
    return f(e, row2d, zeros)


def kernel(x, x_lstm, edge_attr, edge_index,
           ee1_w, ee1_b, ee2_w, ee2_b, ee3_w, ee3_b,
           ne1_w, ne1_b, ne2_w, ne2_b,
           me1_w, me1_b, me2_w, me2_b,
           mn1_w, mn1_b,
           r1_w, r1_b, r2_w, r2_b, r3_w, r3_b, r4_w, r4_b):
    N = x.shape[0]
    E = edge_attr.shape[0]
    D = x_lstm.shape[1]
    T = 3
    assert E % CH == 0 and E % BE == 0 and N % BN == 0 and N % NS == 0

    row2d = edge_index[0].reshape(E // CH, CH)
    col2d = edge_index[1].reshape(E // CH, CH)

    H = me1_w.shape[1]
    w1a, w1b, w1c = me1_w[:D], me1_w[D:2 * D], me1_w[2 * D:]
    am, bm, cm, dm = (mn1_w[:D], mn1_w[D:2 * D],
                      mn1_w[2 * D:3 * D], mn1_w[3 * D:])

    P = 128
    pad_c = lambda w: jnp.pad(w, ((0, 0), (0, P - w.shape[1])))
    pad_v = lambda b: jnp.pad(b, (0, P - b.shape[0]))
    w1a_p, w1b_p = pad_c(w1a), pad_c(w1b)
    w1c_p = jnp.pad(w1c, ((0, P - w1c.shape[0]), (0, P - w1c.shape[1])))
    me2_p = jnp.pad(me2_w, ((0, P - me2_w.shape[0]), (0, P - me2_w.shape[1])))

    h, base, g1, g2 = _node_prep(x, x_lstm, ne1_w, ne1_b, ne2_w, ne2_b,
                                 bm, cm, mn1_b, w1a_p, w1b_p)
    zeros = jnp.zeros((N, D), jnp.float32)
    b1_p, b2_p = pad_v(me1_b), pad_v(me2_b)

    e = None
    out = None
    for t in range(T):
        gs = _sc_gather(g1, g2, row2d, col2d, H)
        if t == 0:
            e = _edge_mlp0(gs, edge_attr, ee1_w, ee1_b, ee2_w, ee2_b,
                           pad_c(ee3_w), pad_v(ee3_b), w1c_p, b1_p,
                           me2_p, b2_p)
        else:
            e = _edge_mlp(gs, e, w1c_p, b1_p, me2_p, b2_p)
        parts = _sc_scatter(e, row2d, zeros, D)
        if t < T - 1:
            h, g1, g2 = _node_update(h, parts[0], parts[1], base, am, dm,
                                     w1a_p, w1b_p)
        else:
            out = _node_final(h, parts[0], parts[1], base, am, dm,
                              r1_w, r1_b, r2_w, r2_b, r3_w, r3_b, r4_w, r4_b)
    return out
